# baseline (speedup 1.0000x reference)
import functools

import jax
import jax.numpy as jnp
from jax import lax
from jax.experimental import pallas as pl
from jax.experimental.pallas import tpu as pltpu
from jax.experimental.pallas import tpu_sc as plsc

E = 16
TOPK = 2
DIM = 2048
INTER = 1024
T = 4096
P = T * TOPK

BM = 128
NUM_BLOCKS = (P + E * (BM - 1) + BM - 1) // BM
PT = NUM_BLOCKS * BM

NC = 2
NS = 16
NW = NC * NS

TW = T // NW
DCT = 8
NDC = TW // DCT
CT = 8
NCT = TW // CT
CLANES = 16
PWL = 128


def _meta_body(idx0_ref, idx1_ref, mask_ref, w0_ref, w1_ref,
               d0_ref, d1_ref, w0r_ref, w1r_ref, be_ref):
    valid0 = mask_ref[...] > 0
    e0 = jnp.where(valid0, idx0_ref[...], -1)
    e1 = jnp.where(valid0, idx1_ref[...], -1)
    e_km = jnp.concatenate([e0, e1], axis=0)
    validf = e_km >= 0
    onehot = (e_km == jax.lax.broadcasted_iota(jnp.int32, (P, E), 1)
              ).astype(jnp.int32)
    cum = onehot
    for s in [1 << k for k in range(13)]:
        cum = cum + jnp.concatenate(
            [jnp.zeros((s, E), jnp.int32), cum[:-s]], axis=0)
    counts = cum[-1:, :]
    rank = jnp.sum(cum * onehot, axis=1, keepdims=True) - 1
    padded = (((counts + BM - 1) // BM) * BM).astype(jnp.float32)
    tri = (jax.lax.broadcasted_iota(jnp.int32, (E, E), 0)
           < jax.lax.broadcasted_iota(jnp.int32, (E, E), 1)
           ).astype(jnp.float32)
    starts = jnp.dot(padded, tri, preferred_element_type=jnp.float32)
    dest_base = jnp.dot(onehot.astype(jnp.float32), starts.reshape(E, 1),
                        preferred_element_type=jnp.float32)
    dest = dest_base.astype(jnp.int32) + rank
    dest = jnp.where(validf, dest, PT - 1)
    d0_ref[...] = dest[:T]
    d1_ref[...] = dest[T:]
    w0r_ref[...] = jnp.broadcast_to(
        jnp.where(valid0, w0_ref[...], 0.0), (T, PWL))
    w1r_ref[...] = jnp.broadcast_to(
        jnp.where(valid0, w1_ref[...], 0.0), (T, PWL))
    bid = jax.lax.broadcasted_iota(jnp.int32, (NUM_BLOCKS, E), 0) * BM
    be_ref[...] = jnp.sum(
        (bid >= starts.astype(jnp.int32)).astype(jnp.int32), axis=1) - 1


def _plan(indices, token_mask, weights):
    mask_i = token_mask[:, None].astype(jnp.int32)
    idx0 = indices[:, 0:1].astype(jnp.int32)
    idx1 = indices[:, 1:2].astype(jnp.int32)
    w0 = weights[:, 0:1]
    w1 = weights[:, 1:2]
    d0, d1, w0r, w1r, be = pl.pallas_call(
        _meta_body,
        out_shape=(
            jax.ShapeDtypeStruct((T, 1), jnp.int32),
            jax.ShapeDtypeStruct((T, 1), jnp.int32),
            jax.ShapeDtypeStruct((T, PWL), jnp.float32),
            jax.ShapeDtypeStruct((T, PWL), jnp.float32),
            jax.ShapeDtypeStruct((NUM_BLOCKS,), jnp.int32),
        ),
    )(idx0, idx1, mask_i, w0, w1)
    return d0, d1, w0r, w1r, be


@functools.lru_cache(maxsize=None)
def _build_dispatch():
    mesh = plsc.VectorSubcoreMesh(core_axis_name="c", subcore_axis_name="s")

    @functools.partial(
        pl.kernel,
        mesh=mesh,
        out_type=(
            jax.ShapeDtypeStruct((PT, DIM), jnp.float32),
            jax.ShapeDtypeStruct((PT, PWL), jnp.float32),
        ),
        scratch_types=[
            pltpu.VMEM((NDC, DCT), jnp.int32),
            pltpu.VMEM((NDC, DCT), jnp.int32),
            pltpu.VMEM((DCT, DIM), jnp.float32),
            pltpu.VMEM((DCT, DIM), jnp.float32),
            pltpu.VMEM((DCT, DIM), jnp.float32),
            pltpu.VMEM((DCT, PWL), jnp.float32),
            pltpu.VMEM((DCT, PWL), jnp.float32),
            pltpu.VMEM((DCT, PWL), jnp.float32),
            pltpu.VMEM((DCT, PWL), jnp.float32),
            pltpu.SemaphoreType.DMA,
            pltpu.SemaphoreType.DMA,
            pltpu.SemaphoreType.DMA,
            pltpu.SemaphoreType.DMA,
            pltpu.SemaphoreType.DMA,
            pltpu.SemaphoreType.DMA,
            pltpu.SemaphoreType.DMA,
            pltpu.SemaphoreType.DMA,
            pltpu.SemaphoreType.DMA,
            pltpu.SemaphoreType.DMA,
            pltpu.SemaphoreType.DMA,
        ],
    )
    def _dispatch(x_hbm, d0_hbm, d1_hbm, w0r_hbm, w1r_hbm,
                  out_hbm, pw_hbm, d0_v, d1_v,
                  buf0, buf1, buf2, wb00, wb01, wb10, wb11,
                  r0, r1, r2, s00, s01, s02, s10, s11, s12, spw0, spw1):
        wid = lax.axis_index("s") * NC + lax.axis_index("c")
        pltpu.sync_copy(d0_hbm.at[wid], d0_v)
        pltpu.sync_copy(d1_hbm.at[wid], d1_v)
        bufs = (buf0, buf1, buf2)
        wb0s, wb1s = (wb00, wb01), (wb10, wb11)
        rsem = (r0, r1, r2)
        s0sem = (s00, s01, s02)
        s1sem = (s10, s11, s12)
        pwsem = (spw0, spw1)
        rcp = [None, None, None]
        scp0 = [None, None, None]
        scp1 = [None, None, None]
        pwcp = [None, None]

        def read(c, b):
            return pltpu.async_copy(
                x_hbm.at[pl.ds(wid * TW + c * DCT, DCT)], bufs[b], rsem[b])

        rcp[0] = read(0, 0)
        rcp[1] = read(1, 1)
        for c in range(NDC):
            p = c % 3
            pp = c & 1
            if c + 2 < NDC:
                q = (c + 2) % 3
                if scp0[q] is not None:
                    scp0[q].wait()
                    scp1[q].wait()
                rcp[q] = read(c + 2, q)
            if pwcp[pp] is not None:
                pwcp[pp].wait()
                pwcp[pp].wait()
            pltpu.sync_copy(
                w0r_hbm.at[pl.ds(wid * TW + c * DCT, DCT)], wb0s[pp])
            pltpu.sync_copy(
                w1r_hbm.at[pl.ds(wid * TW + c * DCT, DCT)], wb1s[pp])
            pwcp[pp] = pltpu.async_copy(
                wb0s[pp], pw_hbm.at[d0_v.at[c]], pwsem[pp])
            pltpu.async_copy(wb1s[pp], pw_hbm.at[d1_v.at[c]], pwsem[pp])
            rcp[p].wait()
            scp0[p] = pltpu.async_copy(
                bufs[p], out_hbm.at[d0_v.at[c]], s0sem[p])
            scp1[p] = pltpu.async_copy(
                bufs[p], out_hbm.at[d1_v.at[c]], s1sem[p])
        for p in range(3):
            if scp0[p] is not None:
                scp0[p].wait()
                scp1[p].wait()
        pwcp[0].wait()
        pwcp[0].wait()
        if pwcp[1] is not None:
            pwcp[1].wait()
            pwcp[1].wait()

    return _dispatch


def _gemm_body(be_ref, a_ref, w1_ref, w2_ref, pw_ref, y_ref):
    a = a_ref[...]
    h = jnp.dot(a, w1_ref[0], preferred_element_type=jnp.float32)
    gate = h[:, :INTER]
    up = h[:, INTER:]
    su = (gate * lax.logistic(gate)) * up * pw_ref[:, :1]
    y_ref[...] = jnp.dot(su, w2_ref[0], preferred_element_type=jnp.float32)


def _grouped_gemm(block_expert, a, w1, w2, pw):
    grid_spec = pltpu.PrefetchScalarGridSpec(
        num_scalar_prefetch=1,
        grid=(NUM_BLOCKS,),
        in_specs=[
            pl.BlockSpec((BM, DIM), lambda i, be: (i, 0)),
            pl.BlockSpec((1, DIM, 2 * INTER), lambda i, be: (be[i], 0, 0)),
            pl.BlockSpec((1, INTER, DIM), lambda i, be: (be[i], 0, 0)),
            pl.BlockSpec((BM, PWL), lambda i, be: (i, 0)),
        ],
        out_specs=pl.BlockSpec((BM, DIM), lambda i, be: (i, 0)),
    )
    return pl.pallas_call(
        _gemm_body,
        grid_spec=grid_spec,
        out_shape=jax.ShapeDtypeStruct((PT, DIM), jnp.float32),
        compiler_params=pltpu.CompilerParams(
            dimension_semantics=("arbitrary",)),
    )(block_expert, a, w1, w2, pw)


@functools.lru_cache(maxsize=None)
def _build_combine():
    mesh = plsc.VectorSubcoreMesh(core_axis_name="c", subcore_axis_name="s")

    @functools.partial(
        pl.kernel,
        mesh=mesh,
        out_type=jax.ShapeDtypeStruct((T, DIM), jnp.float32),
        scratch_types=[
            pltpu.VMEM((NCT, CT), jnp.int32),
            pltpu.VMEM((NCT, CT), jnp.int32),
            pltpu.VMEM((CT, DIM), jnp.float32),
            pltpu.VMEM((CT, DIM), jnp.float32),
            pltpu.VMEM((CT, DIM), jnp.float32),
            pltpu.VMEM((CT, DIM), jnp.float32),
            pltpu.SemaphoreType.DMA,
            pltpu.SemaphoreType.DMA,
            pltpu.SemaphoreType.DMA,
            pltpu.SemaphoreType.DMA,
            pltpu.SemaphoreType.DMA,
            pltpu.SemaphoreType.DMA,
        ],
    )
    def _combine(y_hbm, p0_hbm, p1_hbm, out_hbm,
                 p0_v, p1_v, bufa0, bufa1, bufb0, bufb1,
                 sa0, sa1, sb0, sb1, sw0, sw1):
        wid = lax.axis_index("s") * NC + lax.axis_index("c")
        pltpu.sync_copy(p0_hbm.at[wid], p0_v)
        pltpu.sync_copy(p1_hbm.at[wid], p1_v)
        bufa, bufb = (bufa0, bufa1), (bufb0, bufb1)
        sga, sgb, swb = (sa0, sa1), (sb0, sb1), (sw0, sw1)
        ga = [None, None]
        gb = [None, None]
        wcp = [None, None]
        ga[0] = pltpu.async_copy(y_hbm.at[p0_v.at[0]], bufa0, sa0)
        gb[0] = pltpu.async_copy(y_hbm.at[p1_v.at[0]], bufb0, sb0)
        for c in range(NCT):
            p, q = c & 1, (c + 1) & 1
            if c + 1 < NCT:
                if wcp[q] is not None:
                    wcp[q].wait()
                ga[q] = pltpu.async_copy(
                    y_hbm.at[p0_v.at[c + 1]], bufa[q], sga[q])
                gb[q] = pltpu.async_copy(
                    y_hbm.at[p1_v.at[c + 1]], bufb[q], sgb[q])
            ga[p].wait()
            gb[p].wait()
            ba, bb = bufa[p], bufb[p]

            def vec(j, carry, ba=ba, bb=bb):
                i = j >> 7
                col = pl.multiple_of((j & 127) << 4, 16)
                ba[i, pl.ds(col, 16)] = (
                    ba[i, pl.ds(col, 16)] + bb[i, pl.ds(col, 16)])
                return carry

            lax.fori_loop(0, CT * (DIM // 16), vec, 0, unroll=8)
            wcp[p] = pltpu.async_copy(
                ba, out_hbm.at[pl.ds(wid * TW + c * CT, CT)], swb[p])
        wcp[0].wait()
        wcp[1].wait()

    return _combine


def kernel(x, token_mask, weights, indices, gate_and_up_projs, down_projs):
    d0c, d1c, w0r, w1r, block_expert = _plan(indices, token_mask, weights)
    d0 = d0c.reshape(NW, NDC, DCT)
    d1 = d1c.reshape(NW, NDC, DCT)
    a, pw = _build_dispatch()(x, d0, d1, w0r, w1r)
    y = _grouped_gemm(block_expert, a, gate_and_up_projs, down_projs, pw)
    out = _build_combine()(y, d0, d1)
    return out


# speedup vs baseline: 1.0477x; 1.0477x over previous
import functools

import jax
import jax.numpy as jnp
from jax import lax
from jax.experimental import pallas as pl
from jax.experimental.pallas import tpu as pltpu
from jax.experimental.pallas import tpu_sc as plsc

E = 16
TOPK = 2
DIM = 2048
INTER = 1024
T = 4096
P = T * TOPK

BM = 128
NUM_BLOCKS = (P + E * (BM - 1) + BM - 1) // BM
PT = NUM_BLOCKS * BM

NC = 2
NS = 16
NW = NC * NS

TW = T // NW
DCT = 16
NDC = TW // DCT
CT = 8
NCT = TW // CT
CLANES = 16
PWL = 128


def _meta_body(idx0_ref, idx1_ref, mask_ref, w0_ref, w1_ref,
               d0_ref, d1_ref, w0r_ref, w1r_ref, be_ref):
    valid0 = mask_ref[...] > 0
    e0 = jnp.where(valid0, idx0_ref[...], -1)
    e1 = jnp.where(valid0, idx1_ref[...], -1)
    e_km = jnp.concatenate([e0, e1], axis=0)
    validf = e_km >= 0
    onehot = (e_km == jax.lax.broadcasted_iota(jnp.int32, (P, E), 1)
              ).astype(jnp.int32)
    cum = onehot
    for s in [1 << k for k in range(13)]:
        cum = cum + jnp.concatenate(
            [jnp.zeros((s, E), jnp.int32), cum[:-s]], axis=0)
    counts = cum[-1:, :]
    rank = jnp.sum(cum * onehot, axis=1, keepdims=True) - 1
    padded = (((counts + BM - 1) // BM) * BM).astype(jnp.float32)
    tri = (jax.lax.broadcasted_iota(jnp.int32, (E, E), 0)
           < jax.lax.broadcasted_iota(jnp.int32, (E, E), 1)
           ).astype(jnp.float32)
    starts = jnp.dot(padded, tri, preferred_element_type=jnp.float32)
    dest_base = jnp.dot(onehot.astype(jnp.float32), starts.reshape(E, 1),
                        preferred_element_type=jnp.float32)
    dest = dest_base.astype(jnp.int32) + rank
    dest = jnp.where(validf, dest, PT - 1)
    d0_ref[...] = dest[:T]
    d1_ref[...] = dest[T:]
    w0r_ref[...] = jnp.broadcast_to(
        jnp.where(valid0, w0_ref[...], 0.0), (T, PWL))
    w1r_ref[...] = jnp.broadcast_to(
        jnp.where(valid0, w1_ref[...], 0.0), (T, PWL))
    bid = jax.lax.broadcasted_iota(jnp.int32, (NUM_BLOCKS + 1, E), 0) * BM
    becalc = jnp.sum(
        (bid >= starts.astype(jnp.int32)).astype(jnp.int32), axis=1) - 1
    nreal = (jnp.sum(padded).astype(jnp.int32) + BM - 1) // BM
    be_ref[...] = jnp.where(
        jax.lax.broadcasted_iota(jnp.int32, (NUM_BLOCKS + 1,), 0)
        < NUM_BLOCKS, becalc, nreal)


def _plan(indices, token_mask, weights):
    mask_i = token_mask[:, None].astype(jnp.int32)
    d0, d1, w0r, w1r, be = pl.pallas_call(
        _meta_body,
        out_shape=(
            jax.ShapeDtypeStruct((T, 1), jnp.int32),
            jax.ShapeDtypeStruct((T, 1), jnp.int32),
            jax.ShapeDtypeStruct((T, PWL), jnp.float32),
            jax.ShapeDtypeStruct((T, PWL), jnp.float32),
            jax.ShapeDtypeStruct((NUM_BLOCKS + 1,), jnp.int32),
        ),
    )(indices[:, 0:1].astype(jnp.int32), indices[:, 1:2].astype(jnp.int32),
      mask_i, weights[:, 0:1], weights[:, 1:2])
    return d0, d1, w0r, w1r, be


@functools.lru_cache(maxsize=None)
def _build_dispatch():
    mesh = plsc.VectorSubcoreMesh(core_axis_name="c", subcore_axis_name="s")

    @functools.partial(
        pl.kernel,
        mesh=mesh,
        out_type=(
            jax.ShapeDtypeStruct((PT, DIM), jnp.float32),
            jax.ShapeDtypeStruct((PT, PWL), jnp.float32),
        ),
        scratch_types=[
            pltpu.VMEM((NDC, DCT), jnp.int32),
            pltpu.VMEM((NDC, DCT), jnp.int32),
            pltpu.VMEM((DCT, DIM), jnp.float32),
            pltpu.VMEM((DCT, DIM), jnp.float32),
            pltpu.VMEM((DCT, DIM), jnp.float32),
            pltpu.VMEM((DCT, PWL), jnp.float32),
            pltpu.VMEM((DCT, PWL), jnp.float32),
            pltpu.VMEM((DCT, PWL), jnp.float32),
            pltpu.VMEM((DCT, PWL), jnp.float32),
            pltpu.SemaphoreType.DMA,
            pltpu.SemaphoreType.DMA,
            pltpu.SemaphoreType.DMA,
            pltpu.SemaphoreType.DMA,
            pltpu.SemaphoreType.DMA,
            pltpu.SemaphoreType.DMA,
            pltpu.SemaphoreType.DMA,
            pltpu.SemaphoreType.DMA,
            pltpu.SemaphoreType.DMA,
            pltpu.SemaphoreType.DMA,
            pltpu.SemaphoreType.DMA,
        ],
    )
    def _dispatch(x_hbm, d0_hbm, d1_hbm, w0r_hbm, w1r_hbm,
                  out_hbm, pw_hbm, d0_v, d1_v,
                  buf0, buf1, buf2, wb00, wb01, wb10, wb11,
                  r0, r1, r2, s00, s01, s02, s10, s11, s12, spw0, spw1):
        wid = lax.axis_index("s") * NC + lax.axis_index("c")
        pltpu.sync_copy(d0_hbm.at[wid], d0_v)
        pltpu.sync_copy(d1_hbm.at[wid], d1_v)
        bufs = (buf0, buf1, buf2)
        wb0s, wb1s = (wb00, wb01), (wb10, wb11)
        rsem = (r0, r1, r2)
        s0sem = (s00, s01, s02)
        s1sem = (s10, s11, s12)
        pwsem = (spw0, spw1)
        rcp = [None, None, None]
        scp0 = [None, None, None]
        scp1 = [None, None, None]
        pwcp = [None, None]

        def read(c, b):
            return pltpu.async_copy(
                x_hbm.at[pl.ds(wid * TW + c * DCT, DCT)], bufs[b], rsem[b])

        rcp[0] = read(0, 0)
        rcp[1] = read(1, 1)
        for c in range(NDC):
            p = c % 3
            pp = c & 1
            if c + 2 < NDC:
                q = (c + 2) % 3
                if scp0[q] is not None:
                    scp0[q].wait()
                    scp1[q].wait()
                rcp[q] = read(c + 2, q)
            if pwcp[pp] is not None:
                pwcp[pp].wait()
                pwcp[pp].wait()
            pltpu.sync_copy(
                w0r_hbm.at[pl.ds(wid * TW + c * DCT, DCT)], wb0s[pp])
            pltpu.sync_copy(
                w1r_hbm.at[pl.ds(wid * TW + c * DCT, DCT)], wb1s[pp])
            pwcp[pp] = pltpu.async_copy(
                wb0s[pp], pw_hbm.at[d0_v.at[c]], pwsem[pp])
            pltpu.async_copy(wb1s[pp], pw_hbm.at[d1_v.at[c]], pwsem[pp])
            rcp[p].wait()
            scp0[p] = pltpu.async_copy(
                bufs[p], out_hbm.at[d0_v.at[c]], s0sem[p])
            scp1[p] = pltpu.async_copy(
                bufs[p], out_hbm.at[d1_v.at[c]], s1sem[p])
        for p in range(3):
            if scp0[p] is not None:
                scp0[p].wait()
                scp1[p].wait()
        pwcp[0].wait()
        pwcp[0].wait()
        if pwcp[1] is not None:
            pwcp[1].wait()
            pwcp[1].wait()

    return _dispatch


def _gemm_body(be_ref, a_ref, w1_ref, w2_ref, pw_ref, y_ref):
    @pl.when(pl.program_id(0) < be_ref[NUM_BLOCKS])
    def _():
        a = a_ref[...]
        h = jnp.dot(a, w1_ref[0], preferred_element_type=jnp.float32)
        gate = h[:, :INTER]
        up = h[:, INTER:]
        su = (gate * lax.logistic(gate)) * up * pw_ref[:, :1]
        y_ref[...] = jnp.dot(su, w2_ref[0],
                             preferred_element_type=jnp.float32)


def _grouped_gemm(block_expert, a, w1, w2, pw):
    grid_spec = pltpu.PrefetchScalarGridSpec(
        num_scalar_prefetch=1,
        grid=(NUM_BLOCKS,),
        in_specs=[
            pl.BlockSpec((BM, DIM), lambda i, be: (i, 0)),
            pl.BlockSpec((1, DIM, 2 * INTER), lambda i, be: (be[i], 0, 0)),
            pl.BlockSpec((1, INTER, DIM), lambda i, be: (be[i], 0, 0)),
            pl.BlockSpec((BM, PWL), lambda i, be: (i, 0)),
        ],
        out_specs=pl.BlockSpec((BM, DIM), lambda i, be: (i, 0)),
    )
    return pl.pallas_call(
        _gemm_body,
        grid_spec=grid_spec,
        out_shape=jax.ShapeDtypeStruct((PT, DIM), jnp.float32),
        compiler_params=pltpu.CompilerParams(
            dimension_semantics=("arbitrary",)),
    )(block_expert, a, w1, w2, pw)


@functools.lru_cache(maxsize=None)
def _build_combine():
    mesh = plsc.VectorSubcoreMesh(core_axis_name="c", subcore_axis_name="s")

    @functools.partial(
        pl.kernel,
        mesh=mesh,
        out_type=jax.ShapeDtypeStruct((T, DIM), jnp.float32),
        scratch_types=[
            pltpu.VMEM((NCT, CT), jnp.int32),
            pltpu.VMEM((NCT, CT), jnp.int32),
            pltpu.VMEM((CT, DIM), jnp.float32),
            pltpu.VMEM((CT, DIM), jnp.float32),
            pltpu.VMEM((CT, DIM), jnp.float32),
            pltpu.VMEM((CT, DIM), jnp.float32),
            pltpu.SemaphoreType.DMA,
            pltpu.SemaphoreType.DMA,
            pltpu.SemaphoreType.DMA,
            pltpu.SemaphoreType.DMA,
            pltpu.SemaphoreType.DMA,
            pltpu.SemaphoreType.DMA,
        ],
    )
    def _combine(y_hbm, p0_hbm, p1_hbm, out_hbm,
                 p0_v, p1_v, bufa0, bufa1, bufb0, bufb1,
                 sa0, sa1, sb0, sb1, sw0, sw1):
        wid = lax.axis_index("s") * NC + lax.axis_index("c")
        pltpu.sync_copy(p0_hbm.at[wid], p0_v)
        pltpu.sync_copy(p1_hbm.at[wid], p1_v)
        bufa, bufb = (bufa0, bufa1), (bufb0, bufb1)
        sga, sgb, swb = (sa0, sa1), (sb0, sb1), (sw0, sw1)
        ga = [None, None]
        gb = [None, None]
        wcp = [None, None]
        ga[0] = pltpu.async_copy(y_hbm.at[p0_v.at[0]], bufa0, sa0)
        gb[0] = pltpu.async_copy(y_hbm.at[p1_v.at[0]], bufb0, sb0)
        for c in range(NCT):
            p, q = c & 1, (c + 1) & 1
            if c + 1 < NCT:
                if wcp[q] is not None:
                    wcp[q].wait()
                ga[q] = pltpu.async_copy(
                    y_hbm.at[p0_v.at[c + 1]], bufa[q], sga[q])
                gb[q] = pltpu.async_copy(
                    y_hbm.at[p1_v.at[c + 1]], bufb[q], sgb[q])
            ga[p].wait()
            gb[p].wait()
            ba, bb = bufa[p], bufb[p]

            def vec(j, carry, ba=ba, bb=bb):
                i = j >> 7
                col = pl.multiple_of((j & 127) << 4, 16)
                ba[i, pl.ds(col, 16)] = (
                    ba[i, pl.ds(col, 16)] + bb[i, pl.ds(col, 16)])
                return carry

            lax.fori_loop(0, CT * (DIM // 16), vec, 0, unroll=8)
            wcp[p] = pltpu.async_copy(
                ba, out_hbm.at[pl.ds(wid * TW + c * CT, CT)], swb[p])
        wcp[0].wait()
        wcp[1].wait()

    return _combine


def kernel(x, token_mask, weights, indices, gate_and_up_projs, down_projs):
    d0c, d1c, w0r, w1r, block_expert = _plan(indices, token_mask, weights)
    d0 = d0c.reshape(NW, NDC, DCT)
    d1 = d1c.reshape(NW, NDC, DCT)
    a, pw = _build_dispatch()(x, d0, d1, w0r, w1r)
    y = _grouped_gemm(block_expert, a, gate_and_up_projs, down_projs, pw)
    p0 = d0c.reshape(NW, NCT, CT)
    p1 = d1c.reshape(NW, NCT, CT)
    out = _build_combine()(y, p0, p1)
    return out
